# baseline (speedup 1.0000x reference)
import functools

import jax
import jax.numpy as jnp
from jax import lax
from jax.experimental import pallas as pl
from jax.experimental.pallas import tpu as pltpu
from jax.experimental.pallas import tpu_sc as plsc

_A1 = 10001
_B1 = 160001
_FA = 128
_FB = 144
_H = 256
_MAXNB = 16
_NMOL = 250
_MOLSZ = 40
_DEPTH = 3

_AP = 10240
_BP = 163840
_NW = 32
_NC = 2

_RB = 512
_CA = 4
_CB = 64

_RBF = 640
_NMOLP = 256
_ROWSP = _NMOLP * _MOLSZ


def _mm_in_body(x_ref, w_ref, bi_ref, bm_ref):
    z = jnp.dot(x_ref[...], w_ref[...], preferred_element_type=jnp.float32)
    bi_ref[...] = z
    bm_ref[...] = jnp.maximum(z, 0.0)


def _mm_in(f_bonds_p, w_i):
    return pl.pallas_call(
        _mm_in_body,
        grid=(_BP // _RB,),
        in_specs=[
            pl.BlockSpec((_RB, _FB), lambda i: (i, 0)),
            pl.BlockSpec((_FB, _H), lambda i: (0, 0)),
        ],
        out_specs=[
            pl.BlockSpec((_RB, _H), lambda i: (i, 0)),
            pl.BlockSpec((_RB, _H), lambda i: (i, 0)),
        ],
        out_shape=[jax.ShapeDtypeStruct((_BP, _H), jnp.float32)] * 2,
    )(f_bonds_p, w_i)


def _mm_h_body(x_ref, w_ref, skip_ref, o_ref):
    z = jnp.dot(x_ref[...], w_ref[...], preferred_element_type=jnp.float32)
    o_ref[...] = jnp.maximum(skip_ref[...] + z, 0.0)


def _mm_h(m, w_h, b_input):
    return pl.pallas_call(
        _mm_h_body,
        grid=(_BP // _RB,),
        in_specs=[
            pl.BlockSpec((_RB, _H), lambda i: (i, 0)),
            pl.BlockSpec((_H, _H), lambda i: (0, 0)),
            pl.BlockSpec((_RB, _H), lambda i: (i, 0)),
        ],
        out_specs=pl.BlockSpec((_RB, _H), lambda i: (i, 0)),
        out_shape=jax.ShapeDtypeStruct((_BP, _H), jnp.float32),
    )(m, w_h, b_input)


def _final_body(x1_ref, x2_ref, wa_ref, wb_ref, b_ref, o_ref):
    h = jnp.dot(x1_ref[...], wa_ref[...], preferred_element_type=jnp.float32)
    h = h + jnp.dot(x2_ref[...], wb_ref[...], preferred_element_type=jnp.float32)
    h = jnp.maximum(h + b_ref[...], 0.0)
    o_ref[...] = jnp.sum(h.reshape(_RBF // _MOLSZ, _MOLSZ, _H), axis=1) * (
        1.0 / _MOLSZ)


def _final(f_atoms1, a_msg1, wo_a, wo_b, b_o):
    return pl.pallas_call(
        _final_body,
        grid=(_ROWSP // _RBF,),
        in_specs=[
            pl.BlockSpec((_RBF, _FA), lambda i: (i, 0)),
            pl.BlockSpec((_RBF, _H), lambda i: (i, 0)),
            pl.BlockSpec((_FA, _H), lambda i: (0, 0)),
            pl.BlockSpec((_H, _H), lambda i: (0, 0)),
            pl.BlockSpec((1, _H), lambda i: (0, 0)),
        ],
        out_specs=pl.BlockSpec((_RBF // _MOLSZ, _H), lambda i: (i, 0)),
        out_shape=jax.ShapeDtypeStruct((_NMOLP, _H), jnp.float32),
    )(f_atoms1, a_msg1, wo_a, wo_b, b_o)


def _sc_mesh():
    return plsc.VectorSubcoreMesh(core_axis_name="c", subcore_axis_name="s")


def _gather_sum_body(bmsg, a2bf, out, idx_v, rows_v, acc_v, sem):
    wid = lax.axis_index("s") * _NC + lax.axis_index("c")
    apw = _AP // _NW
    base = wid * apw

    def chunk(ci, carry):
        a0 = base + ci * _CA
        pltpu.sync_copy(a2bf.at[pl.ds(a0 * _MAXNB, _CA * _MAXNB)], idx_v)
        pltpu.async_copy(bmsg.at[idx_v], rows_v, sem).wait()
        for a in range(_CA):
            for cc in range(_H // 16):
                sl = pl.ds(cc * 16, 16)
                v = rows_v[a * _MAXNB, sl]
                for k in range(1, _MAXNB):
                    v = v + rows_v[a * _MAXNB + k, sl]
                acc_v[a, sl] = v
        pltpu.sync_copy(acc_v, out.at[pl.ds(a0, _CA)])
        return carry

    lax.fori_loop(0, apw // _CA, chunk, 0)


def _gather_sum(b_message, a2b_flat):
    k = functools.partial(
        pl.kernel,
        out_type=jax.ShapeDtypeStruct((_AP, _H), jnp.float32),
        mesh=_sc_mesh(),
        scratch_types=[
            pltpu.VMEM((_CA * _MAXNB,), jnp.int32),
            pltpu.VMEM((_CA * _MAXNB, _H), jnp.float32),
            pltpu.VMEM((_CA, _H), jnp.float32),
            pltpu.SemaphoreType.DMA,
        ],
    )(_gather_sum_body)
    return k(b_message, a2b_flat)


def _bond_update_body(amsg, bmsg, b2a, b2revb, out, ia_v, ir_v, ar_v, rr_v,
                      sem_a, sem_r):
    wid = lax.axis_index("s") * _NC + lax.axis_index("c")
    bpw = _BP // _NW
    base = wid * bpw

    def chunk(ci, carry):
        b0 = base + ci * _CB
        pltpu.sync_copy(b2a.at[pl.ds(b0, _CB)], ia_v)
        pltpu.sync_copy(b2revb.at[pl.ds(b0, _CB)], ir_v)
        cpa = pltpu.async_copy(amsg.at[ia_v], ar_v, sem_a)
        cpr = pltpu.async_copy(bmsg.at[ir_v], rr_v, sem_r)
        cpa.wait()
        cpr.wait()
        for r in range(_CB):
            for cc in range(_H // 16):
                sl = pl.ds(cc * 16, 16)
                ar_v[r, sl] = ar_v[r, sl] - rr_v[r, sl]
        pltpu.sync_copy(ar_v, out.at[pl.ds(b0, _CB)])
        return carry

    lax.fori_loop(0, bpw // _CB, chunk, 0)


def _bond_update(a_message, b_message, b2a_p, b2revb_p):
    k = functools.partial(
        pl.kernel,
        out_type=jax.ShapeDtypeStruct((_BP, _H), jnp.float32),
        mesh=_sc_mesh(),
        scratch_types=[
            pltpu.VMEM((_CB,), jnp.int32),
            pltpu.VMEM((_CB,), jnp.int32),
            pltpu.VMEM((_CB, _H), jnp.float32),
            pltpu.VMEM((_CB, _H), jnp.float32),
            pltpu.SemaphoreType.DMA,
            pltpu.SemaphoreType.DMA,
        ],
    )(_bond_update_body)
    return k(a_message, b_message, b2a_p, b2revb_p)


def kernel(f_atoms, f_bonds, a2b, b2a, b2revb, a_scope, W_i, W_h, W_o, b_o):
    f_bonds_p = jnp.pad(f_bonds, ((0, _BP - _B1), (0, 0)))
    a2b_flat = jnp.pad(a2b, ((0, _AP - _A1), (0, 0))).reshape(-1)
    b2a_p = jnp.pad(b2a, (0, _BP - _B1))
    b2revb_p = jnp.pad(b2revb, (0, _BP - _B1))

    b_input, b_message = _mm_in(f_bonds_p, W_i)
    for _ in range(_DEPTH - 1):
        a_message = _gather_sum(b_message, a2b_flat)
        m = _bond_update(a_message, b_message, b2a_p, b2revb_p)
        b_message = _mm_h(m, W_h, b_input)
    a_message = _gather_sum(b_message, a2b_flat)

    f_atoms1 = jnp.pad(f_atoms[1:_A1], ((0, _ROWSP - (_A1 - 1)), (0, 0)))
    a_msg1 = jnp.pad(a_message[1:_A1], ((0, _ROWSP - (_A1 - 1)), (0, 0)))
    wo_a = W_o[:_FA]
    wo_b = W_o[_FA:]
    mols = _final(f_atoms1, a_msg1, wo_a, wo_b, b_o.reshape(1, _H))
    return mols[:_NMOL]


# speedup vs baseline: 1.7426x; 1.7426x over previous
import functools

import jax
import jax.numpy as jnp
from jax import lax
from jax.experimental import pallas as pl
from jax.experimental.pallas import tpu as pltpu
from jax.experimental.pallas import tpu_sc as plsc

_A1 = 10001
_B1 = 160001
_FA = 128
_FB = 144
_H = 256
_MAXNB = 16
_NMOL = 250
_MOLSZ = 40
_DEPTH = 3

_AP = 10240
_ZROW = 10000
_BP = 163840
_NW = 32
_NC = 2

_RB = 512
_CA = 8
_CB = 32

_RBF = 640
_NMOLP = 256
_ROWSP = _NMOLP * _MOLSZ


def _mm_in_body(x_ref, w_ref, bi_ref, bm_ref):
    z = jnp.dot(x_ref[...], w_ref[...], preferred_element_type=jnp.float32)
    bi_ref[...] = z
    bm_ref[...] = jnp.maximum(z, 0.0)


def _mm_in(f_bonds, w_i):
    return pl.pallas_call(
        _mm_in_body,
        grid=(_BP // _RB,),
        in_specs=[
            pl.BlockSpec((_RB, _FB),
                         lambda i: (jnp.minimum(i, (_B1 - 1) // _RB), 0)),
            pl.BlockSpec((_FB, _H), lambda i: (0, 0)),
        ],
        out_specs=[
            pl.BlockSpec((_RB, _H), lambda i: (i, 0)),
            pl.BlockSpec((_RB, _H), lambda i: (i, 0)),
        ],
        out_shape=[jax.ShapeDtypeStruct((_BP, _H), jnp.float32)] * 2,
    )(f_bonds, w_i)


def _mm_h_body(x_ref, w_ref, skip_ref, o_ref):
    z = jnp.dot(x_ref[...], w_ref[...], preferred_element_type=jnp.float32)
    o_ref[...] = jnp.maximum(skip_ref[...] + z, 0.0)


def _mm_h(m, w_h, b_input):
    return pl.pallas_call(
        _mm_h_body,
        grid=(_BP // _RB,),
        in_specs=[
            pl.BlockSpec((_RB, _H), lambda i: (i, 0)),
            pl.BlockSpec((_H, _H), lambda i: (0, 0)),
            pl.BlockSpec((_RB, _H), lambda i: (i, 0)),
        ],
        out_specs=pl.BlockSpec((_RB, _H), lambda i: (i, 0)),
        out_shape=jax.ShapeDtypeStruct((_BP, _H), jnp.float32),
    )(m, w_h, b_input)


def _final_body(x1_ref, x2_ref, wa_ref, wb_ref, b_ref, o_ref):
    h = jnp.dot(x1_ref[...], wa_ref[...], preferred_element_type=jnp.float32)
    h = h + jnp.dot(x2_ref[...], wb_ref[...], preferred_element_type=jnp.float32)
    h = jnp.maximum(h + b_ref[...], 0.0)
    o_ref[...] = jnp.sum(h.reshape(_RBF // _MOLSZ, _MOLSZ, _H), axis=1) * (
        1.0 / _MOLSZ)


def _final(f_atoms1, a_msg1, wo_a, wo_b, b_o):
    return pl.pallas_call(
        _final_body,
        grid=(_ROWSP // _RBF,),
        in_specs=[
            pl.BlockSpec((_RBF, _FA), lambda i: (i, 0)),
            pl.BlockSpec((_RBF, _H), lambda i: (i, 0)),
            pl.BlockSpec((_FA, _H), lambda i: (0, 0)),
            pl.BlockSpec((_H, _H), lambda i: (0, 0)),
            pl.BlockSpec((1, _H), lambda i: (0, 0)),
        ],
        out_specs=pl.BlockSpec((_RBF // _MOLSZ, _H), lambda i: (i, 0)),
        out_shape=jax.ShapeDtypeStruct((_NMOLP, _H), jnp.float32),
    )(f_atoms1, a_msg1, wo_a, wo_b, b_o)


def _sc_mesh():
    return plsc.VectorSubcoreMesh(core_axis_name="c", subcore_axis_name="s")


def _gather_sum_body(bmsg, a2bf, out, idx_v, rows_v, acc_v, sem_g0, sem_g1,
                     sem_s0, sem_s1):
    sem_g = (sem_g0, sem_g1)
    sem_s = (sem_s0, sem_s1)
    wid = lax.axis_index("s") * _NC + lax.axis_index("c")
    apw = _AP // _NW
    base = wid * apw
    nchunks = apw // _CA
    gbytes = _CA * _MAXNB * _H * 4

    pltpu.sync_copy(a2bf.at[pl.ds(base * _MAXNB, apw * _MAXNB)], idx_v)

    def start_gather(ci, b):
        pltpu.async_copy(
            bmsg.at[idx_v.at[pl.ds(ci * _CA * _MAXNB, _CA * _MAXNB)]],
            rows_v.at[b], sem_g[b])

    for b in range(2):
        start_gather(b, b)

    @pl.loop(0, nchunks, step=2)
    def _group(g):
        for b in range(2):
            ci = g + b
            a0 = base + ci * _CA
            pltpu.make_async_copy(
                bmsg.at[idx_v.at[pl.ds(0, _CA * _MAXNB)]], rows_v.at[b],
                sem_g[b]).wait()

            @pl.when(ci >= 2)
            def _drain():
                pltpu.make_async_copy(
                    acc_v.at[b], out.at[pl.ds(base, _CA)], sem_s[b]).wait()

            for a in range(_CA):
                for cc in range(_H // 16):
                    sl = pl.ds(cc * 16, 16)
                    v = rows_v[b, a * _MAXNB, sl]
                    for k in range(1, _MAXNB):
                        v = v + rows_v[b, a * _MAXNB + k, sl]
                    acc_v[b, a, sl] = v
            pltpu.async_copy(acc_v.at[b], out.at[pl.ds(a0, _CA)], sem_s[b])

            @pl.when(ci + 2 < nchunks)
            def _next():
                start_gather(ci + 2, b)

    for b in range(2):
        pltpu.make_async_copy(
            acc_v.at[b], out.at[pl.ds(base, _CA)], sem_s[b]).wait()


def _gather_sum(b_message, a2b_flat):
    k = functools.partial(
        pl.kernel,
        out_type=jax.ShapeDtypeStruct((_AP, _H), jnp.float32),
        mesh=_sc_mesh(),
        scratch_types=[
            pltpu.VMEM((_AP // _NW * _MAXNB,), jnp.int32),
            pltpu.VMEM((2, _CA * _MAXNB, _H), jnp.float32),
            pltpu.VMEM((2, _CA, _H), jnp.float32),
            pltpu.SemaphoreType.DMA,
            pltpu.SemaphoreType.DMA,
            pltpu.SemaphoreType.DMA,
            pltpu.SemaphoreType.DMA,
        ],
    )(_gather_sum_body)
    return k(b_message, a2b_flat)


def _bond_update_body(amsg, bmsg, b2a, b2revb, out, ia_v, ir_v, ar_v, rr_v,
                      o_v, sem_a0, sem_a1, sem_r0, sem_r1, sem_s0, sem_s1):
    sem_a = (sem_a0, sem_a1)
    sem_r = (sem_r0, sem_r1)
    sem_s = (sem_s0, sem_s1)
    wid = lax.axis_index("s") * _NC + lax.axis_index("c")
    bpw = _BP // _NW
    base = wid * bpw
    nchunks = bpw // _CB

    pltpu.sync_copy(b2a.at[pl.ds(base, bpw)], ia_v)
    pltpu.sync_copy(b2revb.at[pl.ds(base, bpw)], ir_v)

    def start_gathers(ci, b):
        sl = pl.ds(ci * _CB, _CB)
        pltpu.async_copy(amsg.at[ia_v.at[sl]], ar_v.at[b], sem_a[b])
        pltpu.async_copy(bmsg.at[ir_v.at[sl]], rr_v.at[b], sem_r[b])

    for b in range(2):
        start_gathers(b, b)

    @pl.loop(0, nchunks, step=2)
    def _group(g):
        for b in range(2):
            ci = g + b
            b0 = base + ci * _CB
            sl0 = pl.ds(0, _CB)
            pltpu.make_async_copy(
                amsg.at[ia_v.at[sl0]], ar_v.at[b], sem_a[b]).wait()
            pltpu.make_async_copy(
                bmsg.at[ir_v.at[sl0]], rr_v.at[b], sem_r[b]).wait()

            @pl.when(ci >= 2)
            def _drain():
                pltpu.make_async_copy(
                    o_v.at[b], out.at[pl.ds(base, _CB)], sem_s[b]).wait()

            for r in range(_CB):
                for cc in range(_H // 16):
                    sl = pl.ds(cc * 16, 16)
                    o_v[b, r, sl] = ar_v[b, r, sl] - rr_v[b, r, sl]
            pltpu.async_copy(o_v.at[b], out.at[pl.ds(b0, _CB)], sem_s[b])

            @pl.when(ci + 2 < nchunks)
            def _next():
                start_gathers(ci + 2, b)

    for b in range(2):
        pltpu.make_async_copy(
            o_v.at[b], out.at[pl.ds(base, _CB)], sem_s[b]).wait()


def _bond_update(a_message, b_message, b2a_s, b2revb_p):
    k = functools.partial(
        pl.kernel,
        out_type=jax.ShapeDtypeStruct((_BP, _H), jnp.float32),
        mesh=_sc_mesh(),
        scratch_types=[
            pltpu.VMEM((_BP // _NW,), jnp.int32),
            pltpu.VMEM((_BP // _NW,), jnp.int32),
            pltpu.VMEM((2, _CB, _H), jnp.float32),
            pltpu.VMEM((2, _CB, _H), jnp.float32),
            pltpu.VMEM((2, _CB, _H), jnp.float32),
            pltpu.SemaphoreType.DMA,
            pltpu.SemaphoreType.DMA,
            pltpu.SemaphoreType.DMA,
            pltpu.SemaphoreType.DMA,
            pltpu.SemaphoreType.DMA,
            pltpu.SemaphoreType.DMA,
        ],
    )(_bond_update_body)
    return k(a_message, b_message, b2a_s, b2revb_p)


def kernel(f_atoms, f_bonds, a2b, b2a, b2revb, a_scope, W_i, W_h, W_o, b_o):
    a2b_flat = jnp.pad(a2b[1:], ((0, _AP - _A1 + 1), (0, 0))).reshape(-1)
    b2a_s = jnp.pad(jnp.where(b2a == 0, _ZROW, b2a - 1),
                    (0, _BP - _B1), constant_values=_ZROW)
    b2revb_p = jnp.pad(b2revb, (0, _BP - _B1))

    b_input, b_message = _mm_in(f_bonds, W_i)
    for _ in range(_DEPTH - 1):
        a_message = _gather_sum(b_message, a2b_flat)
        m = _bond_update(a_message, b_message, b2a_s, b2revb_p)
        b_message = _mm_h(m, W_h, b_input)
    a_message = _gather_sum(b_message, a2b_flat)

    f_atoms1 = jnp.pad(f_atoms[1:_A1], ((0, _ROWSP - (_A1 - 1)), (0, 0)))
    wo_a = W_o[:_FA]
    wo_b = W_o[_FA:]
    mols = _final(f_atoms1, a_message, wo_a, wo_b, b_o.reshape(1, _H))
    return mols[:_NMOL]
